# baseline (speedup 1.0000x reference)
import jax
import jax.numpy as jnp
from jax import lax
from jax.experimental import pallas as pl
from jax.experimental.pallas import tpu as pltpu
from jax.experimental.pallas import tpu_sc as plsc

_INFO = plsc.get_sparse_core_info()
_NC = _INFO.num_cores
_NS = _INFO.num_subcores
_L = _INFO.num_lanes
_NW = _NC * _NS

_BATCH = 16384
_D = 32
_BPW = _BATCH // _NW


def _collab_body(users_hbm, movies_hbm, eu_hbm, em_hbm, bu_hbm, bm_hbm,
                 out_hbm, idx_u, idx_m, rows_u, rows_m, bu_v, bm_v, out_v,
                 sem):
    wid = lax.axis_index("s") * _NC + lax.axis_index("c")
    base = wid * _BPW

    pltpu.sync_copy(users_hbm.at[pl.ds(base, _BPW)], idx_u)
    pltpu.sync_copy(movies_hbm.at[pl.ds(base, _BPW)], idx_m)

    cp1 = pltpu.async_copy(eu_hbm.at[idx_u], rows_u, sem)
    cp2 = pltpu.async_copy(em_hbm.at[idx_m], rows_m, sem)
    cp3 = pltpu.async_copy(bu_hbm.at[idx_u], bu_v, sem)
    cp4 = pltpu.async_copy(bm_hbm.at[idx_m], bm_v, sem)
    cp1.wait()
    cp2.wait()
    cp3.wait()
    cp4.wait()

    lanes = lax.iota(jnp.int32, _L)
    onehots = [lanes == k for k in range(_L)]
    shuf8 = (lanes + 8) % _L
    shuf4 = (lanes + 4) % _L
    shuf2 = (lanes + 2) % _L
    shuf1 = (lanes + 1) % _L

    dnums = lax.GatherDimensionNumbers(
        offset_dims=(), collapsed_slice_dims=(0,), start_index_map=(0,))

    def shuffle(t, idx):
        return lax.gather(t, idx[:, None], dnums, slice_sizes=(1,),
                          mode=lax.GatherScatterMode.PROMISE_IN_BOUNDS)

    def chunk(c, carry):
        b = c * _L
        dot = bu_v[pl.ds(b, _L)] + bm_v[pl.ds(b, _L)]
        for k in range(_L):
            r = b + k
            t = (rows_u[r, pl.ds(0, _L)] * rows_m[r, pl.ds(0, _L)] +
                 rows_u[r, pl.ds(_L, _L)] * rows_m[r, pl.ds(_L, _L)])
            t = t + shuffle(t, shuf8)
            t = t + shuffle(t, shuf4)
            t = t + shuffle(t, shuf2)
            t = t + shuffle(t, shuf1)
            dot = dot + jnp.where(onehots[k], t, 0.0)
        out_v[pl.ds(b, _L)] = 5.25 / (1.0 + jnp.exp(-dot))
        return carry

    lax.fori_loop(0, _BPW // _L, chunk, 0)
    pltpu.sync_copy(out_v, out_hbm.at[pl.ds(base, _BPW)])


def kernel(users, movies, embedding_user, embedding_movie, bias_user,
           bias_movie):
    mesh = plsc.VectorSubcoreMesh(core_axis_name="c", subcore_axis_name="s")
    run = pl.kernel(
        _collab_body,
        mesh=mesh,
        compiler_params=pltpu.CompilerParams(use_tc_tiling_on_sc=False),
        out_type=jax.ShapeDtypeStruct((_BATCH,), jnp.float32),
        scratch_types=[
            pltpu.VMEM((_BPW,), jnp.int32),
            pltpu.VMEM((_BPW,), jnp.int32),
            pltpu.VMEM((_BPW, _D), jnp.float32),
            pltpu.VMEM((_BPW, _D), jnp.float32),
            pltpu.VMEM((_BPW,), jnp.float32),
            pltpu.VMEM((_BPW,), jnp.float32),
            pltpu.VMEM((_BPW,), jnp.float32),
            pltpu.SemaphoreType.DMA,
        ],
    )
    one = lax.optimization_barrier(jnp.float32(1.0))
    return run(users.astype(jnp.int32), movies.astype(jnp.int32),
               embedding_user * one, embedding_movie * one,
               bias_user, bias_movie)


# speedup vs baseline: 2.5912x; 2.5912x over previous
import jax
import jax.numpy as jnp
from jax import lax
from jax.experimental import pallas as pl
from jax.experimental.pallas import tpu as pltpu
from jax.experimental.pallas import tpu_sc as plsc

_INFO = plsc.get_sparse_core_info()
_NC = _INFO.num_cores
_NS = _INFO.num_subcores
_L = _INFO.num_lanes
_NW = _NC * _NS

_BATCH = 16384
_D = 32
_BPW = _BATCH // _NW
_NU = 1000000


def _lane_helpers():
    lanes = lax.iota(jnp.int32, _L)
    onehots = [lanes == k for k in range(_L)]
    dnums = lax.GatherDimensionNumbers(
        offset_dims=(), collapsed_slice_dims=(0,), start_index_map=(0,))

    def shuffle(t, idx):
        return lax.gather(t, idx[:, None], dnums, slice_sizes=(1,),
                          mode=lax.GatherScatterMode.PROMISE_IN_BOUNDS)

    return lanes, onehots, shuffle


def _user_body(users_hbm, eu_hbm, bu_hbm, urows_hbm, buv_hbm,
               idx_u, rows_u, rows1d, buw, buv_v, semrow, semb):
    wid = lax.axis_index("s") * _NC + lax.axis_index("c")
    base = wid * _BPW

    pltpu.sync_copy(users_hbm.at[pl.ds(base, _BPW)], idx_u)

    def issue(g, carry):
        b = g * _L
        su16 = idx_u[pl.ds(b, _L)]
        for k in range(_L):
            r = b + k
            pltpu.async_copy(
                eu_hbm.at[pl.ds(su16[k], 1)],
                rows_u.at[pl.ds(r, 1)], semrow)
            st = pl.multiple_of(jnp.minimum(su16[k] & ~7, _NU - _L), 8)
            pltpu.async_copy(
                bu_hbm.at[pl.ds(st, _L)],
                buw.at[pl.ds(pl.multiple_of(r * _L, 8), _L)], semb)
        return carry

    lax.fori_loop(0, _BPW // _L, issue, 0)

    pltpu.make_async_copy(bu_hbm.at[pl.ds(0, _BPW * _L)], buw, semb).wait()

    lanes, onehots, shuffle = _lane_helpers()

    def bias_sel(g, carry):
        b = g * _L
        su16 = idx_u[pl.ds(b, _L)]
        posv = su16 - jnp.minimum(su16 & ~7, _NU - _L)
        acc = jnp.zeros((_L,), jnp.float32)
        for k in range(_L):
            w = buw[pl.ds(pl.multiple_of((b + k) * _L, 8), _L)]
            val = shuffle(w, jnp.full((_L,), posv[k], jnp.int32))
            acc = acc + jnp.where(onehots[k], val, 0.0)
        buv_v[pl.ds(b, _L)] = acc
        return carry

    lax.fori_loop(0, _BPW // _L, bias_sel, 0)

    pltpu.make_async_copy(eu_hbm.at[pl.ds(0, _BPW)], rows_u,
                          semrow).wait()

    def relayout(r, carry):
        o = pl.multiple_of(r * _D, 8)
        rows1d[pl.ds(o, _L)] = rows_u[r, pl.ds(0, _L)]
        rows1d[pl.ds(o + _L, _L)] = rows_u[r, pl.ds(_L, _L)]
        return carry

    lax.fori_loop(0, _BPW, relayout, 0)
    pltpu.sync_copy(rows1d, urows_hbm.at[pl.ds(base * _D, _BPW * _D)])
    pltpu.sync_copy(buv_v, buv_hbm.at[pl.ds(base, _BPW)])


def _movie_body(movies_hbm, em_hbm, bm_hbm, urows_hbm, buv_hbm, out_hbm,
                idx_m, rows_m, urows1d, bmv_v, buv_v, out_v, semrow, sem):
    wid = lax.axis_index("s") * _NC + lax.axis_index("c")
    base = wid * _BPW

    pltpu.sync_copy(movies_hbm.at[pl.ds(base, _BPW)], idx_m)

    cpb = pltpu.async_copy(bm_hbm.at[idx_m], bmv_v, sem)
    cpu = pltpu.async_copy(urows_hbm.at[pl.ds(base * _D, _BPW * _D)],
                           urows1d, sem)
    cpv = pltpu.async_copy(buv_hbm.at[pl.ds(base, _BPW)], buv_v, sem)

    def issue(g, carry):
        b = g * _L
        sm16 = idx_m[pl.ds(b, _L)]
        for k in range(_L):
            r = b + k
            pltpu.async_copy(
                em_hbm.at[pl.ds(sm16[k], 1)],
                rows_m.at[pl.ds(r, 1)], semrow)
        return carry

    lax.fori_loop(0, _BPW // _L, issue, 0)

    cpb.wait()
    cpu.wait()
    cpv.wait()
    pltpu.make_async_copy(em_hbm.at[pl.ds(0, _BPW)], rows_m,
                          semrow).wait()

    lanes, onehots, shuffle = _lane_helpers()
    shuf8 = (lanes + 8) % _L
    shuf4 = (lanes + 4) % _L
    shuf2 = (lanes + 2) % _L
    shuf1 = (lanes + 1) % _L

    def chunk(c, carry):
        b = c * _L
        dot = bmv_v[pl.ds(b, _L)] + buv_v[pl.ds(b, _L)]
        for k in range(_L):
            r = b + k
            o = pl.multiple_of(r * _D, 8)
            t = (urows1d[pl.ds(o, _L)] * rows_m[r, pl.ds(0, _L)] +
                 urows1d[pl.ds(o + _L, _L)] * rows_m[r, pl.ds(_L, _L)])
            t = t + shuffle(t, shuf8)
            t = t + shuffle(t, shuf4)
            t = t + shuffle(t, shuf2)
            t = t + shuffle(t, shuf1)
            dot = dot + jnp.where(onehots[k], t, 0.0)
        out_v[pl.ds(b, _L)] = 5.25 / (1.0 + jnp.exp(-dot))
        return carry

    lax.fori_loop(0, _BPW // _L, chunk, 0)

    pltpu.sync_copy(out_v, out_hbm.at[pl.ds(base, _BPW)])


def kernel(users, movies, embedding_user, embedding_movie, bias_user,
           bias_movie):
    mesh = plsc.VectorSubcoreMesh(core_axis_name="c", subcore_axis_name="s")

    k_user = pl.kernel(
        _user_body,
        mesh=mesh,
        out_type=(jax.ShapeDtypeStruct((_BATCH * _D,), jnp.float32),
                  jax.ShapeDtypeStruct((_BATCH,), jnp.float32)),
        scratch_types=[
            pltpu.VMEM((_BPW,), jnp.int32),
            pltpu.VMEM((_BPW, _D), jnp.float32),
            pltpu.VMEM((_BPW * _D,), jnp.float32),
            pltpu.VMEM((_BPW * _L,), jnp.float32),
            pltpu.VMEM((_BPW,), jnp.float32),
            pltpu.SemaphoreType.DMA,
            pltpu.SemaphoreType.DMA,
        ],
    )
    urows, buv = k_user(users.astype(jnp.int32), embedding_user, bias_user)

    k_movie = pl.kernel(
        _movie_body,
        mesh=mesh,
        out_type=jax.ShapeDtypeStruct((_BATCH,), jnp.float32),
        scratch_types=[
            pltpu.VMEM((_BPW,), jnp.int32),
            pltpu.VMEM((_BPW, _D), jnp.float32),
            pltpu.VMEM((_BPW * _D,), jnp.float32),
            pltpu.VMEM((_BPW,), jnp.float32),
            pltpu.VMEM((_BPW,), jnp.float32),
            pltpu.VMEM((_BPW,), jnp.float32),
            pltpu.SemaphoreType.DMA,
            pltpu.SemaphoreType.DMA,
        ],
    )
    return k_movie(movies.astype(jnp.int32), embedding_movie, bias_movie,
                   urows, buv)
